# Initial kernel scaffold; baseline (speedup 1.0000x reference)
import jax
import jax.numpy as jnp
from jax.experimental import pallas as pl


def kernel(latents, codebook):
    raise NotImplementedError("write your pallas kernel here")


# speedup vs baseline: 1.2621x; 1.2621x over previous
import functools

import jax
import jax.numpy as jnp
from jax.experimental import pallas as pl
from jax.experimental.pallas import tpu as pltpu

BETA_P1 = 1.25


def _vq_body(x_ref, cb_ref, cbt_ref, q_ref, loss_ref):
    b = pl.program_id(0)
    xb = x_ref[0]
    cb = cb_ref[...]
    C, D = cb.shape
    HW = xb.shape[1]

    x_sq = jnp.sum(xb * xb, axis=0, keepdims=True)
    cb_sq = jnp.sum(cb * cb, axis=1, keepdims=True)
    mm = jax.lax.dot_general(
        cb, xb, (((1,), (0,)), ((), ())),
        preferred_element_type=jnp.float32)
    dist = (x_sq + cb_sq) - 2.0 * mm

    minv = jnp.min(dist, axis=0, keepdims=True)
    iota = jax.lax.broadcasted_iota(jnp.int32, (C, HW), 0).astype(jnp.float32)
    masked = jnp.where(dist == minv, iota, jnp.float32(C))
    idx = jnp.min(masked, axis=0, keepdims=True)
    onehot = (iota == idx).astype(jnp.float32)

    q = jax.lax.dot_general(
        cbt_ref[...], onehot, (((1,), (0,)), ((), ())),
        preferred_element_type=jnp.float32,
        precision=jax.lax.Precision.HIGHEST)
    q_ref[0] = q

    @pl.when(b == 0)
    def _():
        loss_ref[0, 0] = 0.0
    loss_ref[0, 0] += jnp.sum(minv)


def kernel(latents, codebook):
    B, D, H, W = latents.shape
    C = codebook.shape[0]
    HW = H * W
    x = latents.reshape(B, D, HW)
    cbt = codebook.T

    q, s = pl.pallas_call(
        _vq_body,
        grid=(B,),
        in_specs=[
            pl.BlockSpec((1, D, HW), lambda b: (b, 0, 0)),
            pl.BlockSpec((C, D), lambda b: (0, 0)),
            pl.BlockSpec((D, C), lambda b: (0, 0)),
        ],
        out_specs=[
            pl.BlockSpec((1, D, HW), lambda b: (b, 0, 0)),
            pl.BlockSpec(memory_space=pltpu.SMEM),
        ],
        out_shape=[
            jax.ShapeDtypeStruct((B, D, HW), jnp.float32),
            jax.ShapeDtypeStruct((1, 1), jnp.float32),
        ],
    )(x, codebook, cbt)

    vq_loss = (BETA_P1 / (B * HW * D)) * s[0, 0]
    return (q.reshape(B, D, H, W), vq_loss)


# speedup vs baseline: 2.1070x; 1.6694x over previous
import jax
import jax.numpy as jnp
from jax.experimental import pallas as pl
from jax.experimental.pallas import tpu as pltpu

BETA_P1 = 1.25


def _vq_body(x_ref, cb_ref, cbt_ref, q_ref, loss_ref):
    b = pl.program_id(0)
    xb = x_ref[0]
    cb = cb_ref[...]

    C = cb_ref.shape[0]
    x_sq = jnp.sum(xb * xb, axis=0, keepdims=True)
    cb_sq = jnp.sum(cb * cb, axis=1, keepdims=True)
    mm = jax.lax.dot_general(
        cb, xb, (((1,), (0,)), ((), ())),
        preferred_element_type=jnp.float32)
    dist = (x_sq + cb_sq) - 2.0 * mm

    minv = jnp.min(dist, axis=0, keepdims=True)
    iota_c = jax.lax.broadcasted_iota(jnp.int32, (C, 1), 0).astype(jnp.float32)
    masked = jnp.where(dist == minv, iota_c, jnp.float32(C))
    idx = jnp.min(masked, axis=0, keepdims=True)
    onehot = (iota_c == idx).astype(jnp.float32)

    q = jax.lax.dot_general(
        cbt_ref[...], onehot, (((1,), (0,)), ((), ())),
        preferred_element_type=jnp.float32)
    q_ref[0] = q

    @pl.when(b == 0)
    def _():
        loss_ref[0, 0] = 0.0
    loss_ref[0, 0] += jnp.sum(minv)


def kernel(latents, codebook):
    B, D, H, W = latents.shape
    C = codebook.shape[0]
    HW = H * W
    x = latents.reshape(B, D, HW)
    cbt = codebook.T

    q, s = pl.pallas_call(
        _vq_body,
        grid=(B,),
        in_specs=[
            pl.BlockSpec((1, D, HW), lambda b: (b, 0, 0)),
            pl.BlockSpec((C, D), lambda b: (0, 0)),
            pl.BlockSpec((D, C), lambda b: (0, 0)),
        ],
        out_specs=[
            pl.BlockSpec((1, D, HW), lambda b: (b, 0, 0)),
            pl.BlockSpec(memory_space=pltpu.SMEM),
        ],
        out_shape=[
            jax.ShapeDtypeStruct((B, D, HW), jnp.float32),
            jax.ShapeDtypeStruct((1, 1), jnp.float32),
        ],
    )(x, codebook, cbt)

    vq_loss = (BETA_P1 / (B * HW * D)) * s[0, 0]
    return (q.reshape(B, D, H, W), vq_loss)


# speedup vs baseline: 2.1534x; 1.0220x over previous
import jax
import jax.numpy as jnp
from jax.experimental import pallas as pl
from jax.experimental.pallas import tpu as pltpu

BETA_P1 = 1.25


def _vq_body(x_ref, cb_ref, cbt_ref, q_ref, loss_ref):
    b = pl.program_id(0)
    xb = x_ref[0]
    cb = cb_ref[...]

    C = cb_ref.shape[0]
    x_sq = jnp.sum(xb * xb, axis=0, keepdims=True)
    cb_sq = jnp.sum(cb * cb, axis=1, keepdims=True)
    mm = jax.lax.dot_general(
        cb, xb, (((1,), (0,)), ((), ())),
        preferred_element_type=jnp.float32)
    dist = (0.5 * x_sq + 0.5 * cb_sq) - mm

    minv = jnp.min(dist, axis=0, keepdims=True)
    iota_c = jax.lax.broadcasted_iota(jnp.int32, (C, 1), 0).astype(jnp.float32)
    masked = jnp.where(dist == minv, iota_c, jnp.float32(C))
    idx = jnp.min(masked, axis=0, keepdims=True)
    onehot = (iota_c == idx).astype(jnp.float32)

    q = jax.lax.dot_general(
        cbt_ref[...], onehot, (((1,), (0,)), ((), ())),
        preferred_element_type=jnp.float32)
    q_ref[0] = q

    @pl.when(b == 0)
    def _():
        loss_ref[0, 0] = 0.0
    loss_ref[0, 0] += jnp.sum(minv)


def kernel(latents, codebook):
    B, D, H, W = latents.shape
    C = codebook.shape[0]
    HW = H * W
    x = latents.reshape(B, D, HW)
    cbt = codebook.T

    q, s = pl.pallas_call(
        _vq_body,
        grid=(B,),
        in_specs=[
            pl.BlockSpec((1, D, HW), lambda b: (b, 0, 0)),
            pl.BlockSpec((C, D), lambda b: (0, 0)),
            pl.BlockSpec((D, C), lambda b: (0, 0)),
        ],
        out_specs=[
            pl.BlockSpec((1, D, HW), lambda b: (b, 0, 0)),
            pl.BlockSpec(memory_space=pltpu.SMEM),
        ],
        out_shape=[
            jax.ShapeDtypeStruct((B, D, HW), jnp.float32),
            jax.ShapeDtypeStruct((1, 1), jnp.float32),
        ],
    )(x, codebook, cbt)

    vq_loss = (2.0 * BETA_P1 / (B * HW * D)) * s[0, 0]
    return (q.reshape(B, D, H, W), vq_loss)


# speedup vs baseline: 2.1837x; 1.0141x over previous
import jax
import jax.numpy as jnp
from jax.experimental import pallas as pl
from jax.experimental.pallas import tpu as pltpu

BETA_P1 = 1.25


def _vq_body(x_ref, cb_ref, cbt_ref, q_ref, loss_ref):
    b = pl.program_id(0)
    cb = cb_ref[...]
    C = cb_ref.shape[0]
    G = x_ref.shape[0]

    cb_sq = jnp.sum(cb * cb, axis=1, keepdims=True)
    iota_c = jax.lax.broadcasted_iota(jnp.int32, (C, 1), 0).astype(jnp.float32)

    @pl.when(b == 0)
    def _():
        loss_ref[0, 0] = 0.0

    for g in range(G):
        xb = x_ref[g]
        x_sq = jnp.sum(xb * xb, axis=0, keepdims=True)
        mm = jax.lax.dot_general(
            cb, xb, (((1,), (0,)), ((), ())),
            preferred_element_type=jnp.float32)
        dist = (0.5 * x_sq + 0.5 * cb_sq) - mm

        minv = jnp.min(dist, axis=0, keepdims=True)
        masked = jnp.where(dist == minv, iota_c, jnp.float32(C))
        idx = jnp.min(masked, axis=0, keepdims=True)
        onehot = (iota_c == idx).astype(jnp.float32)

        q = jax.lax.dot_general(
            cbt_ref[...], onehot, (((1,), (0,)), ((), ())),
            preferred_element_type=jnp.float32)
        q_ref[g] = q
        loss_ref[0, 0] += jnp.sum(minv)


def kernel(latents, codebook):
    B, D, H, W = latents.shape
    C = codebook.shape[0]
    HW = H * W
    x = latents.reshape(B, D, HW)
    cbt = codebook.T

    G = 2
    q, s = pl.pallas_call(
        _vq_body,
        grid=(B // G,),
        in_specs=[
            pl.BlockSpec((G, D, HW), lambda b: (b, 0, 0)),
            pl.BlockSpec((C, D), lambda b: (0, 0)),
            pl.BlockSpec((D, C), lambda b: (0, 0)),
        ],
        out_specs=[
            pl.BlockSpec((G, D, HW), lambda b: (b, 0, 0)),
            pl.BlockSpec(memory_space=pltpu.SMEM),
        ],
        out_shape=[
            jax.ShapeDtypeStruct((B, D, HW), jnp.float32),
            jax.ShapeDtypeStruct((1, 1), jnp.float32),
        ],
    )(x, codebook, cbt)

    vq_loss = (2.0 * BETA_P1 / (B * HW * D)) * s[0, 0]
    return (q.reshape(B, D, H, W), vq_loss)
